# baseline (speedup 1.0000x reference)
import functools

import jax
import jax.numpy as jnp
from jax import lax
from jax.experimental import pallas as pl
from jax.experimental.pallas import tpu as pltpu
from jax.experimental.pallas import tpu_sc as plsc

N_MAIN = 100000
N_AUX = 10000
DIM = 64
NC = 2
NS = 16
NW = NC * NS
QB = 128


@functools.lru_cache(maxsize=None)
def _make_kernel(batch, hist):
    b_per_w = batch // NW
    i_per_w = b_per_w * hist
    n_bhi = b_per_w // 128
    n_units = (b_per_w // QB) * hist
    assert batch % (NW * QB) == 0 and b_per_w % 128 == 0
    mesh = plsc.VectorSubcoreMesh(core_axis_name="c", subcore_axis_name="s")

    @functools.partial(
        pl.kernel,
        mesh=mesh,
        out_type=jax.ShapeDtypeStruct((hist, DIM // 8, batch // 128, 8, 128),
                                      jnp.float32),
        scratch_types=[
            pltpu.VMEM((i_per_w,), jnp.int32),
            pltpu.VMEM((hist, b_per_w), jnp.int32),
            pltpu.VMEM((2, QB, 128), jnp.float32),
            pltpu.VMEM((2, DIM, 129), jnp.float32),
            pltpu.SemaphoreType.DMA,
            pltpu.SemaphoreType.DMA,
            pltpu.SemaphoreType.DMA,
            pltpu.SemaphoreType.DMA,
        ],
        compiler_params=pltpu.CompilerParams(use_tc_tiling_on_sc=False,
                                             needs_layout_passes=False),
    )
    def gather_kernel(idx_hbm, table_hbm, out_hbm,
                      idx_v, idx_t, rows_v, tbuf, gsem0, gsem1, ssem0, ssem1):
        cid = lax.axis_index("c")
        sid = lax.axis_index("s")
        wid = sid * NC + cid
        bhi0 = wid * n_bhi

        pltpu.sync_copy(idx_hbm.at[pl.ds(wid * i_per_w, i_per_w)], idx_v)
        iota = lax.iota(jnp.int32, 16)
        iota_h = iota * hist

        def build_t(k, carry):
            h = k // (b_per_w // 16)
            b0 = (k % (b_per_w // 16)) * 16
            v = plsc.load_gather(idx_v, [b0 * hist + iota_h + h])
            idx_t[h, pl.ds(b0, 16)] = v
            return carry

        lax.fori_loop(0, hist * (b_per_w // 16), build_t, 0)

        gsems = (gsem0, gsem1)
        ssems = (ssem0, ssem1)

        def gstart(u, slot):
            pltpu.async_copy(
                table_hbm.at[idx_t.at[u % hist].at[pl.ds((u // hist) * QB,
                                                         QB)]],
                rows_v.at[slot],
                gsems[slot],
            )

        def gwait(slot):
            pltpu.make_async_copy(
                table_hbm.at[idx_t.at[0].at[pl.ds(0, QB)]],
                rows_v.at[slot],
                gsems[slot],
            ).wait()

        dvecs = [dblk * 16 + iota for dblk in range(DIM // 16)]
        zeros16 = iota * 0

        def transpose(slot):
            @plsc.parallel_loop(0, QB, unroll=8)
            def _(blo):
                blo_v = zeros16 + blo
                for dblk in range(DIM // 16):
                    v = rows_v[slot, blo, pl.ds(dblk * 16, 16)]
                    plsc.store_scatter(
                        tbuf.at[slot],
                        [dvecs[dblk], blo_v],
                        v,
                    )

        def sstart(u, slot):
            h = u % hist
            hb = u // hist
            for d_hi in range(DIM // 8):
                pltpu.async_copy(
                    tbuf.at[slot].at[pl.ds(d_hi * 8, 8), pl.ds(0, 128)],
                    out_hbm.at[h, d_hi, bhi0 + hb],
                    ssems[slot],
                )

        def sdrain(slot):
            for d_hi in range(DIM // 8):
                pltpu.make_async_copy(
                    tbuf.at[slot].at[pl.ds(d_hi * 8, 8), pl.ds(0, 128)],
                    out_hbm.at[0, d_hi, bhi0],
                    ssems[slot],
                ).wait()

        gstart(0, 0)

        def body2(i, carry):
            u0 = 2 * i
            u1 = u0 + 1
            gstart(u1, 1)
            gwait(0)

            @pl.when(i > 0)
            def _():
                sdrain(0)

            transpose(0)
            sstart(u0, 0)
            @pl.when(u1 + 1 < n_units)
            def _():
                gstart(u1 + 1, 0)

            gwait(1)

            @pl.when(i > 0)
            def _():
                sdrain(1)

            transpose(1)
            sstart(u1, 1)
            return carry

        lax.fori_loop(0, n_units // 2, body2, 0)
        sdrain(0)
        sdrain(1)

    return gather_kernel


def kernel(x, W_main, W_aux):
    batch, hist = x.shape
    table = jnp.pad(jnp.concatenate([W_main, W_aux], axis=0),
                    ((0, 0), (0, 128 - DIM)))
    idx = x.reshape(batch * hist)
    out5 = _make_kernel(batch, hist)(idx, table)
    return out5.transpose(2, 4, 0, 1, 3).reshape(batch, hist, DIM)


# speedup vs baseline: 1.2228x; 1.2228x over previous
import functools

import jax
import jax.numpy as jnp
from jax import lax
from jax.experimental import pallas as pl
from jax.experimental.pallas import tpu as pltpu
from jax.experimental.pallas import tpu_sc as plsc

N_MAIN = 100000
N_AUX = 10000
DIM = 64
NC = 2
NS = 16
NW = NC * NS


@functools.lru_cache(maxsize=None)
def _make_kernel(batch, hist):
    b_per_w = batch // NW
    i_per_w = b_per_w * hist
    n_bhi = b_per_w // 128
    half = b_per_w // 2
    n_units = 2 * hist
    assert batch % (NW * 256) == 0
    mesh = plsc.VectorSubcoreMesh(core_axis_name="c", subcore_axis_name="s")

    @functools.partial(
        pl.kernel,
        mesh=mesh,
        out_type=jax.ShapeDtypeStruct((hist, DIM // 8, batch // 128, 8, 128),
                                      jnp.float32),
        scratch_types=[
            pltpu.VMEM((hist, b_per_w), jnp.int32),
            pltpu.VMEM((2, half, DIM), jnp.float32),
            pltpu.VMEM((2, 2, DIM, 129), jnp.float32),
            pltpu.SemaphoreType.DMA,
            pltpu.SemaphoreType.DMA,
            pltpu.SemaphoreType.DMA,
            pltpu.SemaphoreType.DMA,
        ],
        compiler_params=pltpu.CompilerParams(use_tc_tiling_on_sc=False,
                                             needs_layout_passes=False),
    )
    def gather_kernel(xt_hbm, table_hbm, out_hbm,
                      idx_t, rows_v, tbuf, gsem0, gsem1, ssem0, ssem1):
        cid = lax.axis_index("c")
        sid = lax.axis_index("s")
        wid = sid * NC + cid
        bhi0 = wid * n_bhi

        pltpu.sync_copy(xt_hbm.at[:, pl.ds(wid * b_per_w, b_per_w)], idx_t)
        iota = lax.iota(jnp.int32, 16)

        gsems = (gsem0, gsem1)
        ssems = (ssem0, ssem1)

        def gstart(u, slot):
            pltpu.async_copy(
                table_hbm.at[idx_t.at[u % hist].at[pl.ds((u // hist) * half,
                                                         half)]],
                rows_v.at[slot],
                gsems[slot],
            )

        def gwait(slot):
            pltpu.make_async_copy(
                table_hbm.at[idx_t.at[0].at[pl.ds(0, half)]],
                rows_v.at[slot],
                gsems[slot],
            ).wait()

        dvecs = [dblk * 16 + iota for dblk in range(DIM // 16)]
        zeros16 = iota * 0

        def transpose(slot):
            for bhi in range(half // 128):
                base = bhi * 128
                bhi_v = zeros16 + bhi

                @plsc.parallel_loop(0, 128, unroll=8)
                def _(blo):
                    blo_v = zeros16 + blo
                    for dblk in range(DIM // 16):
                        v = rows_v[slot, base + blo, pl.ds(dblk * 16, 16)]
                        plsc.store_scatter(
                            tbuf.at[slot],
                            [bhi_v, dvecs[dblk], blo_v],
                            v,
                        )

        def sstart(u, slot):
            h = u % hist
            hb = (u // hist) * 2
            for d_hi in range(DIM // 8):
                pltpu.async_copy(
                    tbuf.at[slot].at[:, pl.ds(d_hi * 8, 8), pl.ds(0, 128)],
                    out_hbm.at[h, d_hi].at[pl.ds(bhi0 + hb, 2)],
                    ssems[slot],
                )

        def sdrain(slot):
            for d_hi in range(DIM // 8):
                pltpu.make_async_copy(
                    tbuf.at[slot].at[:, pl.ds(d_hi * 8, 8), pl.ds(0, 128)],
                    out_hbm.at[0, d_hi].at[pl.ds(bhi0, 2)],
                    ssems[slot],
                ).wait()

        gstart(0, 0)

        def body2(i, carry):
            u0 = 2 * i
            u1 = u0 + 1
            gstart(u1, 1)
            gwait(0)

            @pl.when(i > 0)
            def _():
                sdrain(0)

            transpose(0)
            sstart(u0, 0)
            @pl.when(u1 + 1 < n_units)
            def _():
                gstart(u1 + 1, 0)

            gwait(1)

            @pl.when(i > 0)
            def _():
                sdrain(1)

            transpose(1)
            sstart(u1, 1)
            return carry

        lax.fori_loop(0, n_units // 2, body2, 0)
        sdrain(0)
        sdrain(1)

    return gather_kernel


def kernel(x, W_main, W_aux):
    batch, hist = x.shape
    table = jnp.concatenate([W_main, W_aux], axis=0)
    out5 = _make_kernel(batch, hist)(x.T, table)
    return out5.transpose(2, 4, 0, 1, 3).reshape(batch, hist, DIM)
